# Initial kernel scaffold; baseline (speedup 1.0000x reference)
import jax
import jax.numpy as jnp
from jax.experimental import pallas as pl


def kernel(x, router_w, gate_up_w, down_w):
    raise NotImplementedError("write your pallas kernel here")


# speedup vs baseline: 1.3242x; 1.3242x over previous
import functools

import jax
import jax.numpy as jnp
from jax.experimental import pallas as pl
from jax.experimental.pallas import tpu as pltpu

D_MODEL = 1024
D_FF = 4096
E = 16
TOP_K = 2
N_TOK = 64
T_FF = 512


def _moe_kernel(x_ref, rw_ref, gw_ref, uw_ref, dw_ref, out_ref, gates_ref):
    e = pl.program_id(0)
    t = pl.program_id(1)

    @pl.when((e == 0) & (t == 0))
    def _init():
        logits = jax.lax.dot_general(
            x_ref[...], rw_ref[...], (((1,), (1,)), ((), ())),
            preferred_element_type=jnp.float32)
        l1 = jnp.max(logits, axis=-1, keepdims=True)
        i1 = jnp.argmax(logits, axis=-1)[:, None]
        eids = jax.lax.broadcasted_iota(jnp.int32, (N_TOK, E), 1)
        masked = jnp.where(eids == i1, -jnp.inf, logits)
        l2 = jnp.max(masked, axis=-1, keepdims=True)
        i2 = jnp.argmax(masked, axis=-1)[:, None]
        w1 = jax.nn.sigmoid(l1 - l2)
        w2 = 1.0 - w1
        gates = jnp.where(eids == i1, w1, 0.0) + jnp.where(eids == i2, w2, 0.0)
        gates_ref[...] = gates.T
        out_ref[...] = jnp.zeros_like(out_ref)

    g = gates_ref[e, :]
    xe = x_ref[...] * g[:, None]
    gate = jax.lax.dot_general(
        xe, gw_ref[0], (((1,), (1,)), ((), ())),
        preferred_element_type=jnp.float32)
    up = jax.lax.dot_general(
        xe, uw_ref[0], (((1,), (1,)), ((), ())),
        preferred_element_type=jnp.float32)
    gate = jnp.clip(gate, -10.0, 10.0)
    hidden = jax.nn.silu(gate) * up
    out_ref[...] += jax.lax.dot_general(
        hidden, dw_ref[0], (((1,), (1,)), ((), ())),
        preferred_element_type=jnp.float32)


@functools.partial(jax.jit, static_argnames=("interpret",))
def kernel(x, router_w, gate_up_w, down_w, interpret=False):
    n_t = D_FF // T_FF
    grid = (E, n_t)
    return pl.pallas_call(
        _moe_kernel,
        grid=grid,
        in_specs=[
            pl.BlockSpec((N_TOK, D_MODEL), lambda e, t: (0, 0)),
            pl.BlockSpec((E, D_MODEL), lambda e, t: (0, 0)),
            pl.BlockSpec((1, T_FF, D_MODEL), lambda e, t: (e, t, 0)),
            pl.BlockSpec((1, T_FF, D_MODEL), lambda e, t: (e, t + D_FF // T_FF, 0)),
            pl.BlockSpec((1, D_MODEL, T_FF), lambda e, t: (e, 0, t)),
        ],
        out_specs=pl.BlockSpec((N_TOK, D_MODEL), lambda e, t: (0, 0)),
        out_shape=jax.ShapeDtypeStruct((N_TOK, D_MODEL), jnp.float32),
        scratch_shapes=[pltpu.VMEM((E, N_TOK), jnp.float32)],
        compiler_params=pltpu.CompilerParams(
            dimension_semantics=("arbitrary", "arbitrary")),
        interpret=interpret,
    )(x, router_w, gate_up_w, gate_up_w, down_w)


# speedup vs baseline: 1.4841x; 1.1208x over previous
import functools

import jax
import jax.numpy as jnp
from jax.experimental import pallas as pl
from jax.experimental.pallas import tpu as pltpu

D_MODEL = 1024
D_FF = 4096
E = 16
TOP_K = 2
N_TOK = 64
T_FF = 1024


def _moe_kernel(x_ref, rw_ref, gw_ref, uw_ref, dw_ref, out_ref, gates_ref):
    e = pl.program_id(0)
    t = pl.program_id(1)

    @pl.when((e == 0) & (t == 0))
    def _init():
        logits = jax.lax.dot_general(
            x_ref[...], rw_ref[...], (((1,), (1,)), ((), ())),
            preferred_element_type=jnp.float32)
        l1 = jnp.max(logits, axis=-1, keepdims=True)
        i1 = jnp.argmax(logits, axis=-1)[:, None]
        eids = jax.lax.broadcasted_iota(jnp.int32, (N_TOK, E), 1)
        masked = jnp.where(eids == i1, -jnp.inf, logits)
        l2 = jnp.max(masked, axis=-1, keepdims=True)
        i2 = jnp.argmax(masked, axis=-1)[:, None]
        w1 = jax.nn.sigmoid(l1 - l2)
        w2 = 1.0 - w1
        gates = jnp.where(eids == i1, w1, 0.0) + jnp.where(eids == i2, w2, 0.0)
        gates_ref[...] = gates.T
        out_ref[...] = jnp.zeros_like(out_ref)

    g = gates_ref[e, :]
    xe = x_ref[...] * g[:, None]
    gate = jax.lax.dot_general(
        xe, gw_ref[0], (((1,), (1,)), ((), ())),
        preferred_element_type=jnp.float32)
    up = jax.lax.dot_general(
        xe, uw_ref[0], (((1,), (1,)), ((), ())),
        preferred_element_type=jnp.float32)
    gate = jnp.clip(gate, -10.0, 10.0)
    hidden = jax.nn.silu(gate) * up
    out_ref[...] += jax.lax.dot_general(
        hidden, dw_ref[0], (((1,), (1,)), ((), ())),
        preferred_element_type=jnp.float32)


@functools.partial(jax.jit, static_argnames=("interpret",))
def kernel(x, router_w, gate_up_w, down_w, interpret=False):
    n_t = D_FF // T_FF
    grid = (E, n_t)
    return pl.pallas_call(
        _moe_kernel,
        grid=grid,
        in_specs=[
            pl.BlockSpec((N_TOK, D_MODEL), lambda e, t: (0, 0)),
            pl.BlockSpec((E, D_MODEL), lambda e, t: (0, 0)),
            pl.BlockSpec((1, T_FF, D_MODEL), lambda e, t: (e, t, 0)),
            pl.BlockSpec((1, T_FF, D_MODEL), lambda e, t: (e, t + D_FF // T_FF, 0)),
            pl.BlockSpec((1, D_MODEL, T_FF), lambda e, t: (e, 0, t)),
        ],
        out_specs=pl.BlockSpec((N_TOK, D_MODEL), lambda e, t: (0, 0)),
        out_shape=jax.ShapeDtypeStruct((N_TOK, D_MODEL), jnp.float32),
        scratch_shapes=[pltpu.VMEM((E, N_TOK), jnp.float32)],
        compiler_params=pltpu.CompilerParams(
            dimension_semantics=("arbitrary", "arbitrary")),
        interpret=interpret,
    )(x, router_w, gate_up_w, gate_up_w, down_w)


# speedup vs baseline: 1.4848x; 1.0005x over previous
import functools

import jax
import jax.numpy as jnp
from jax.experimental import pallas as pl
from jax.experimental.pallas import tpu as pltpu

D_MODEL = 1024
D_FF = 4096
E = 16
TOP_K = 2
N_TOK = 64
T_FF = 2048


def _moe_kernel(x_ref, rw_ref, gw_ref, uw_ref, dw_ref, out_ref, gates_ref):
    e = pl.program_id(0)
    t = pl.program_id(1)

    @pl.when((e == 0) & (t == 0))
    def _init():
        logits = jax.lax.dot_general(
            x_ref[...], rw_ref[...], (((1,), (1,)), ((), ())),
            preferred_element_type=jnp.float32)
        l1 = jnp.max(logits, axis=-1, keepdims=True)
        i1 = jnp.argmax(logits, axis=-1)[:, None]
        eids = jax.lax.broadcasted_iota(jnp.int32, (N_TOK, E), 1)
        masked = jnp.where(eids == i1, -jnp.inf, logits)
        l2 = jnp.max(masked, axis=-1, keepdims=True)
        i2 = jnp.argmax(masked, axis=-1)[:, None]
        w1 = jax.nn.sigmoid(l1 - l2)
        w2 = 1.0 - w1
        gates = jnp.where(eids == i1, w1, 0.0) + jnp.where(eids == i2, w2, 0.0)
        gates_ref[...] = gates.T
        out_ref[...] = jnp.zeros_like(out_ref)

    g = gates_ref[e, :]
    xe = x_ref[...] * g[:, None]
    gate = jax.lax.dot_general(
        xe, gw_ref[0], (((1,), (1,)), ((), ())),
        preferred_element_type=jnp.float32)
    up = jax.lax.dot_general(
        xe, uw_ref[0], (((1,), (1,)), ((), ())),
        preferred_element_type=jnp.float32)
    gate = jnp.clip(gate, -10.0, 10.0)
    hidden = jax.nn.silu(gate) * up
    out_ref[...] += jax.lax.dot_general(
        hidden, dw_ref[0], (((1,), (1,)), ((), ())),
        preferred_element_type=jnp.float32)


@functools.partial(jax.jit, static_argnames=("interpret",))
def kernel(x, router_w, gate_up_w, down_w, interpret=False):
    n_t = D_FF // T_FF
    grid = (E, n_t)
    return pl.pallas_call(
        _moe_kernel,
        grid=grid,
        in_specs=[
            pl.BlockSpec((N_TOK, D_MODEL), lambda e, t: (0, 0)),
            pl.BlockSpec((E, D_MODEL), lambda e, t: (0, 0)),
            pl.BlockSpec((1, T_FF, D_MODEL), lambda e, t: (e, t, 0)),
            pl.BlockSpec((1, T_FF, D_MODEL), lambda e, t: (e, t + D_FF // T_FF, 0)),
            pl.BlockSpec((1, D_MODEL, T_FF), lambda e, t: (e, 0, t)),
        ],
        out_specs=pl.BlockSpec((N_TOK, D_MODEL), lambda e, t: (0, 0)),
        out_shape=jax.ShapeDtypeStruct((N_TOK, D_MODEL), jnp.float32),
        scratch_shapes=[pltpu.VMEM((E, N_TOK), jnp.float32)],
        compiler_params=pltpu.CompilerParams(
            dimension_semantics=("arbitrary", "arbitrary")),
        interpret=interpret,
    )(x, router_w, gate_up_w, gate_up_w, down_w)
